# baseline (speedup 1.0000x reference)
import functools

import jax
import jax.numpy as jnp
from jax import lax
from jax.experimental import pallas as pl
from jax.experimental.pallas import tpu as pltpu
from jax.experimental.pallas import tpu_sc as plsc

_B = 1024
_F = 64

_NC = 2
_NS = 16
_NW = _NC * _NS
_BPW = _B // _NW


def _sc_gather_dot(t1, t2, table, bias1d):
    mesh = plsc.VectorSubcoreMesh(
        core_axis_name="c", subcore_axis_name="s",
        num_cores=_NC, num_subcores=_NS)

    @functools.partial(
        pl.kernel,
        mesh=mesh,
        out_type=[
            jax.ShapeDtypeStruct((_B,), jnp.float32),
            jax.ShapeDtypeStruct((_B,), jnp.float32),
        ],
        scratch_types=[
            pltpu.VMEM((_BPW,), jnp.int32),
            pltpu.VMEM((_BPW,), jnp.int32),
            pltpu.VMEM((_BPW, _F), jnp.float32),
            pltpu.VMEM((_BPW, _F), jnp.float32),
            pltpu.VMEM((_BPW,), jnp.float32),
            pltpu.VMEM((_BPW,), jnp.float32),
            pltpu.VMEM((_BPW,), jnp.float32),
            pltpu.VMEM((_BPW,), jnp.float32),
            pltpu.SemaphoreType.DMA,
        ],
        compiler_params=pltpu.CompilerParams(
            needs_layout_passes=False, use_tc_tiling_on_sc=False),
    )
    def k(t1_hbm, t2_hbm, table_hbm, bias_hbm, dot_hbm, s_hbm,
          idx1_v, idx2_v, rows1_v, rows2_v, b1_v, b2_v, dots_v, s_v, sem):
        wid = lax.axis_index("s") * _NC + lax.axis_index("c")
        base = wid * _BPW
        pltpu.sync_copy(t1_hbm.at[pl.ds(base, _BPW)], idx1_v)
        pltpu.sync_copy(t2_hbm.at[pl.ds(base, _BPW)], idx2_v)
        c1 = pltpu.async_copy(table_hbm.at[idx1_v], rows1_v, sem)
        c2 = pltpu.async_copy(table_hbm.at[idx2_v], rows2_v, sem)
        c3 = pltpu.async_copy(bias_hbm.at[idx1_v], b1_v, sem)
        c4 = pltpu.async_copy(bias_hbm.at[idx2_v], b2_v, sem)
        c1.wait()
        c2.wait()
        c3.wait()
        c4.wait()
        lanes = lax.iota(jnp.int32, 16)
        for g in range(_BPW // 16):
            rows = jnp.full((16,), g * 16, jnp.int32) + lanes
            acc = (plsc.load_gather(rows1_v, [rows, jnp.zeros((16,), jnp.int32)])
                   * plsc.load_gather(rows2_v, [rows, jnp.zeros((16,), jnp.int32)]))
            for f in range(1, _F):
                col = jnp.full((16,), f, jnp.int32)
                acc = acc + (plsc.load_gather(rows1_v, [rows, col])
                             * plsc.load_gather(rows2_v, [rows, col]))
            dots_v[pl.ds(g * 16, 16)] = acc
        for g in range(_BPW // 16):
            sl = pl.ds(g * 16, 16)
            s_v[sl] = b1_v[sl] + b2_v[sl]
        pltpu.sync_copy(dots_v, dot_hbm.at[pl.ds(base, _BPW)])
        pltpu.sync_copy(s_v, s_hbm.at[pl.ds(base, _BPW)])

    return k(t1, t2, table, bias1d)


def _tc_broadcast(dot, s):
    def body(d_ref, s_ref, o_ref):
        o_ref[...] = s_ref[...] + d_ref[...]

    return pl.pallas_call(
        body,
        out_shape=jax.ShapeDtypeStruct((_B, _B), jnp.float32),
    )(dot.reshape(1, _B), s.reshape(_B, 1))


def kernel(token1, token2, token_embedding, bias_embedding):
    t1 = token1.astype(jnp.int32)
    t2 = token2.astype(jnp.int32)
    dot, s = _sc_gather_dot(t1, t2, token_embedding,
                            bias_embedding.reshape(-1))
    return _tc_broadcast(dot, s)


# speedup vs baseline: 9.9750x; 9.9750x over previous
import functools

import jax
import jax.numpy as jnp
from jax import lax
from jax.experimental import pallas as pl
from jax.experimental.pallas import tpu as pltpu
from jax.experimental.pallas import tpu_sc as plsc

_B = 1024
_F = 64
_W = 128
_WB = 128

_NC = 2
_NS = 16
_NW = _NC * _NS
_BPW = _B // _NW


def _sc_gather_dot(t1, t2, tab_t, bias1d):
    mesh = plsc.VectorSubcoreMesh(
        core_axis_name="c", subcore_axis_name="s",
        num_cores=_NC, num_subcores=_NS)

    @functools.partial(
        pl.kernel,
        mesh=mesh,
        out_type=[
            jax.ShapeDtypeStruct((_B,), jnp.float32),
            jax.ShapeDtypeStruct((_B,), jnp.float32),
        ],
        scratch_types=[
            pltpu.VMEM((_BPW,), jnp.int32),
            pltpu.VMEM((_BPW,), jnp.int32),
            pltpu.VMEM((2, _F, _W), jnp.float32),
            pltpu.VMEM((2, _F, _W), jnp.float32),
            pltpu.VMEM((2, 1, _WB), jnp.float32),
            pltpu.VMEM((2, 1, _WB), jnp.float32),
            pltpu.VMEM((_BPW,), jnp.float32),
            pltpu.VMEM((_BPW,), jnp.float32),
            pltpu.SemaphoreType.DMA((2,)),
        ],
        compiler_params=pltpu.CompilerParams(needs_layout_passes=False),
    )
    def k(t1_hbm, t2_hbm, tab_hbm, bias_hbm, dot_hbm, s_hbm,
          idx1_v, idx2_v, e1_v, e2_v, b1_v, b2_v,
          dots_v, s_v, sem):
        wid = lax.axis_index("s") * _NC + lax.axis_index("c")
        base = wid * _BPW
        pltpu.sync_copy(t1_hbm.at[pl.ds(base, _BPW)], idx1_v)
        pltpu.sync_copy(t2_hbm.at[pl.ds(base, _BPW)], idx2_v)
        lanes = lax.iota(jnp.int32, 16)
        last = lanes == 15

        def scalar_at(ref, p):
            chunk = ref[pl.ds((p // 16) * 16, 16)]
            return lax.reduce_max(
                jnp.where(lanes == (p % 16), chunk, 0), axes=(0,))

        def issue(p, slot):
            i1 = scalar_at(idx1_v, p)
            i2 = scalar_at(idx2_v, p)
            w1 = pl.multiple_of(jnp.bitwise_and(i1, -_W), _W)
            w2 = pl.multiple_of(jnp.bitwise_and(i2, -_W), _W)
            wb1 = pl.multiple_of(jnp.bitwise_and(i1, -_WB), _WB)
            wb2 = pl.multiple_of(jnp.bitwise_and(i2, -_WB), _WB)
            return (
                pltpu.async_copy(tab_hbm.at[:, pl.ds(w1, _W)],
                                 e1_v.at[slot], sem.at[slot]),
                pltpu.async_copy(tab_hbm.at[:, pl.ds(w2, _W)],
                                 e2_v.at[slot], sem.at[slot]),
                pltpu.async_copy(bias_hbm.at[:, pl.ds(wb1, _WB)],
                                 b1_v.at[slot], sem.at[slot]),
                pltpu.async_copy(bias_hbm.at[:, pl.ds(wb2, _WB)],
                                 b2_v.at[slot], sem.at[slot]),
            )

        pend = {0: issue(0, 0)}
        for p in range(_BPW):
            slot = p & 1
            if p + 1 < _BPW:
                pend[(p + 1) & 1] = issue(p + 1, (p + 1) & 1)
            for c in pend[slot]:
                c.wait()
            i1 = scalar_at(idx1_v, p)
            i2 = scalar_at(idx2_v, p)
            col1 = jnp.full((16,), jnp.bitwise_and(i1, _W - 1), jnp.int32)
            col2 = jnp.full((16,), jnp.bitwise_and(i2, _W - 1), jnp.int32)
            cb1 = jnp.full((16,), jnp.bitwise_and(i1, _WB - 1), jnp.int32)
            cb2 = jnp.full((16,), jnp.bitwise_and(i2, _WB - 1), jnp.int32)
            acc = None
            for g in range(_F // 16):
                fidx = jnp.full((16,), g * 16, jnp.int32) + lanes
                prod = (plsc.load_gather(e1_v.at[slot], [fidx, col1])
                        * plsc.load_gather(e2_v.at[slot], [fidx, col2]))
                acc = prod if acc is None else acc + prod
            csum = plsc.cumsum(acc)
            pv = jnp.full((16,), p, jnp.int32)
            plsc.store_scatter(dots_v, [pv], csum, mask=last)
            zer = jnp.zeros((16,), jnp.int32)
            sv = (plsc.load_gather(b1_v.at[slot], [zer, cb1])
                  + plsc.load_gather(b2_v.at[slot], [zer, cb2]))
            plsc.store_scatter(s_v, [pv], sv, mask=last)
        pltpu.sync_copy(dots_v, dot_hbm.at[pl.ds(base, _BPW)])
        pltpu.sync_copy(s_v, s_hbm.at[pl.ds(base, _BPW)])

    return k(t1, t2, tab_t, bias1d)


def _tc_broadcast(dot, s):
    def body(d_ref, s_ref, o_ref):
        o_ref[...] = s_ref[...] + d_ref[...]

    return pl.pallas_call(
        body,
        out_shape=jax.ShapeDtypeStruct((_B, _B), jnp.float32),
    )(dot.reshape(1, _B), s.reshape(_B, 1))


def kernel(token1, token2, token_embedding, bias_embedding):
    t1 = token1.astype(jnp.int32)
    t2 = token2.astype(jnp.int32)
    dot, s = _sc_gather_dot(t1, t2, token_embedding.T,
                            bias_embedding.T)
    return _tc_broadcast(dot, s)


# speedup vs baseline: 10.9925x; 1.1020x over previous
import functools

import jax
import jax.numpy as jnp
from jax import lax
from jax.experimental import pallas as pl
from jax.experimental.pallas import tpu as pltpu
from jax.experimental.pallas import tpu_sc as plsc

_B = 1024
_F = 64
_W = 128
_NBUF = 4

_NC = 2
_NS = 16
_NW = _NC * _NS
_BPW = _B // _NW


def _sc_gather_dot(t1, t2, tab_t, bias_t):
    mesh = plsc.VectorSubcoreMesh(
        core_axis_name="c", subcore_axis_name="s",
        num_cores=_NC, num_subcores=_NS)

    @functools.partial(
        pl.kernel,
        mesh=mesh,
        out_type=[
            jax.ShapeDtypeStruct((_B,), jnp.float32),
            jax.ShapeDtypeStruct((_B,), jnp.float32),
        ],
        scratch_types=[
            pltpu.VMEM((_BPW,), jnp.int32),
            pltpu.VMEM((_BPW,), jnp.int32),
            pltpu.VMEM((_NBUF, _F, _W), jnp.float32),
            pltpu.VMEM((_NBUF, _F, _W), jnp.float32),
            pltpu.VMEM((_BPW, _W), jnp.float32),
            pltpu.VMEM((_BPW, _W), jnp.float32),
            pltpu.VMEM((_BPW,), jnp.float32),
            pltpu.VMEM((_BPW,), jnp.float32),
            pltpu.SemaphoreType.DMA((_NBUF,)),
            pltpu.SemaphoreType.DMA,
        ],
        compiler_params=pltpu.CompilerParams(needs_layout_passes=False),
    )
    def k(t1_hbm, t2_hbm, tab_hbm, bias_hbm, dot_hbm, s_hbm,
          idx1_v, idx2_v, e1_v, e2_v, b1_v, b2_v,
          dots_v, s_v, sem, bsem):
        wid = lax.axis_index("s") * _NC + lax.axis_index("c")
        base = wid * _BPW
        pltpu.sync_copy(t1_hbm.at[pl.ds(base, _BPW)], idx1_v)
        pltpu.sync_copy(t2_hbm.at[pl.ds(base, _BPW)], idx2_v)
        lanes = lax.iota(jnp.int32, 16)
        last = lanes == 15

        def scalar_at(ref, p):
            chunk = ref[pl.ds((p // 16) * 16, 16)]
            return lax.reduce_max(
                jnp.where(lanes == (p % 16), chunk, 0), axes=(0,))

        bias_copies = []
        idx_scalars = []
        for p in range(_BPW):
            i1 = scalar_at(idx1_v, p)
            i2 = scalar_at(idx2_v, p)
            idx_scalars.append((i1, i2))
            wb1 = pl.multiple_of(jnp.bitwise_and(i1, -_W), _W)
            wb2 = pl.multiple_of(jnp.bitwise_and(i2, -_W), _W)
            bias_copies.append(pltpu.async_copy(
                bias_hbm.at[0, pl.ds(wb1, _W)], b1_v.at[p], bsem))
            bias_copies.append(pltpu.async_copy(
                bias_hbm.at[0, pl.ds(wb2, _W)], b2_v.at[p], bsem))

        def issue(p, slot):
            i1, i2 = idx_scalars[p]
            w1 = pl.multiple_of(jnp.bitwise_and(i1, -_W), _W)
            w2 = pl.multiple_of(jnp.bitwise_and(i2, -_W), _W)
            return (
                pltpu.async_copy(tab_hbm.at[:, pl.ds(w1, _W)],
                                 e1_v.at[slot], sem.at[slot]),
                pltpu.async_copy(tab_hbm.at[:, pl.ds(w2, _W)],
                                 e2_v.at[slot], sem.at[slot]),
            )

        pend = {}
        for p in range(min(_NBUF - 1, _BPW)):
            pend[p % _NBUF] = issue(p, p % _NBUF)
        for p in range(_BPW):
            slot = p % _NBUF
            nxt = p + _NBUF - 1
            if nxt < _BPW:
                pend[nxt % _NBUF] = issue(nxt, nxt % _NBUF)
            for c in pend[slot]:
                c.wait()
            i1, i2 = idx_scalars[p]
            col1 = jnp.full((16,), jnp.bitwise_and(i1, _W - 1), jnp.int32)
            col2 = jnp.full((16,), jnp.bitwise_and(i2, _W - 1), jnp.int32)
            acc = None
            for g in range(_F // 16):
                fidx = jnp.full((16,), g * 16, jnp.int32) + lanes
                prod = (plsc.load_gather(e1_v.at[slot], [fidx, col1])
                        * plsc.load_gather(e2_v.at[slot], [fidx, col2]))
                acc = prod if acc is None else acc + prod
            csum = plsc.cumsum(acc)
            plsc.store_scatter(dots_v, [jnp.full((16,), p, jnp.int32)],
                               csum, mask=last)
        for c in bias_copies:
            c.wait()
        for g in range(_BPW // 16):
            sl = pl.ds(g * 16, 16)
            pids = jnp.full((16,), g * 16, jnp.int32) + lanes
            cb1 = jnp.bitwise_and(idx1_v[sl], _W - 1)
            cb2 = jnp.bitwise_and(idx2_v[sl], _W - 1)
            s_v[sl] = (plsc.load_gather(b1_v, [pids, cb1])
                       + plsc.load_gather(b2_v, [pids, cb2]))
        pltpu.sync_copy(dots_v, dot_hbm.at[pl.ds(base, _BPW)])
        pltpu.sync_copy(s_v, s_hbm.at[pl.ds(base, _BPW)])

    return k(t1, t2, tab_t, bias_t)


def _tc_broadcast(dot, s):
    def body(d_ref, s_ref, o_ref):
        o_ref[...] = s_ref[...] + d_ref[...]

    return pl.pallas_call(
        body,
        out_shape=jax.ShapeDtypeStruct((_B, _B), jnp.float32),
    )(dot.reshape(1, _B), s.reshape(_B, 1))


def kernel(token1, token2, token_embedding, bias_embedding):
    t1 = token1.astype(jnp.int32)
    t2 = token2.astype(jnp.int32)
    dot, s = _sc_gather_dot(t1, t2, token_embedding.T,
                            bias_embedding.T)
    return _tc_broadcast(dot, s)


# speedup vs baseline: 12.2470x; 1.1141x over previous
import functools

import jax
import jax.numpy as jnp
from jax import lax
from jax.experimental import pallas as pl
from jax.experimental.pallas import tpu as pltpu
from jax.experimental.pallas import tpu_sc as plsc

_B = 1024
_F = 64
_W = 128
_NBUF = 6

_NC = 2
_NS = 16
_NW = _NC * _NS
_BPW = _B // _NW


def _sc_gather_dot(t1, t2, tab_t, bias_t):
    mesh = plsc.VectorSubcoreMesh(
        core_axis_name="c", subcore_axis_name="s",
        num_cores=_NC, num_subcores=_NS)

    @functools.partial(
        pl.kernel,
        mesh=mesh,
        out_type=[
            jax.ShapeDtypeStruct((_B,), jnp.float32),
            jax.ShapeDtypeStruct((_B,), jnp.float32),
        ],
        scratch_types=[
            pltpu.VMEM((_BPW,), jnp.int32),
            pltpu.VMEM((_BPW,), jnp.int32),
            pltpu.VMEM((_NBUF, _F, _W), jnp.float32),
            pltpu.VMEM((_NBUF, _F, _W), jnp.float32),
            pltpu.VMEM((_BPW, _W), jnp.float32),
            pltpu.VMEM((_BPW, _W), jnp.float32),
            pltpu.VMEM((_BPW,), jnp.float32),
            pltpu.VMEM((_BPW,), jnp.float32),
            pltpu.SemaphoreType.DMA((_NBUF,)),
            pltpu.SemaphoreType.DMA,
        ],
        compiler_params=pltpu.CompilerParams(needs_layout_passes=False),
    )
    def k(t1_hbm, t2_hbm, tab_hbm, bias_hbm, dot_hbm, s_hbm,
          idx1_v, idx2_v, e1_v, e2_v, b1_v, b2_v,
          dots_v, s_v, sem, bsem):
        wid = lax.axis_index("s") * _NC + lax.axis_index("c")
        base = wid * _BPW
        pltpu.sync_copy(t1_hbm.at[pl.ds(base, _BPW)], idx1_v)
        pltpu.sync_copy(t2_hbm.at[pl.ds(base, _BPW)], idx2_v)
        lanes = lax.iota(jnp.int32, 16)
        last = lanes == 15

        def scalar_at(ref, p):
            chunk = ref[pl.ds((p // 16) * 16, 16)]
            return lax.reduce_max(
                jnp.where(lanes == (p % 16), chunk, 0), axes=(0,))

        idx_scalars = [(scalar_at(idx1_v, p), scalar_at(idx2_v, p))
                       for p in range(_BPW)]

        def issue(p, slot):
            i1, i2 = idx_scalars[p]
            w1 = pl.multiple_of(jnp.bitwise_and(i1, -_W), _W)
            w2 = pl.multiple_of(jnp.bitwise_and(i2, -_W), _W)
            return (
                pltpu.async_copy(tab_hbm.at[:, pl.ds(w1, _W)],
                                 e1_v.at[slot], sem.at[slot]),
                pltpu.async_copy(tab_hbm.at[:, pl.ds(w2, _W)],
                                 e2_v.at[slot], sem.at[slot]),
            )

        pend = {}
        for p in range(min(_NBUF - 1, _BPW)):
            pend[p % _NBUF] = issue(p, p % _NBUF)
        bias_copies = []
        for p in range(_BPW):
            i1, i2 = idx_scalars[p]
            wb1 = pl.multiple_of(jnp.bitwise_and(i1, -_W), _W)
            wb2 = pl.multiple_of(jnp.bitwise_and(i2, -_W), _W)
            bias_copies.append(pltpu.async_copy(
                bias_hbm.at[0, pl.ds(wb1, _W)], b1_v.at[p], bsem))
            bias_copies.append(pltpu.async_copy(
                bias_hbm.at[0, pl.ds(wb2, _W)], b2_v.at[p], bsem))
        for p in range(_BPW):
            slot = p % _NBUF
            nxt = p + _NBUF - 1
            if nxt < _BPW:
                pend[nxt % _NBUF] = issue(nxt, nxt % _NBUF)
            for c in pend[slot]:
                c.wait()
            i1, i2 = idx_scalars[p]
            col1 = jnp.full((16,), jnp.bitwise_and(i1, _W - 1), jnp.int32)
            col2 = jnp.full((16,), jnp.bitwise_and(i2, _W - 1), jnp.int32)
            acc = None
            for g in range(_F // 16):
                fidx = jnp.full((16,), g * 16, jnp.int32) + lanes
                prod = (plsc.load_gather(e1_v.at[slot], [fidx, col1])
                        * plsc.load_gather(e2_v.at[slot], [fidx, col2]))
                acc = prod if acc is None else acc + prod
            csum = plsc.cumsum(acc)
            plsc.store_scatter(dots_v, [jnp.full((16,), p, jnp.int32)],
                               csum, mask=last)
        for c in bias_copies:
            c.wait()
        for g in range(_BPW // 16):
            sl = pl.ds(g * 16, 16)
            pids = jnp.full((16,), g * 16, jnp.int32) + lanes
            cb1 = jnp.bitwise_and(idx1_v[sl], _W - 1)
            cb2 = jnp.bitwise_and(idx2_v[sl], _W - 1)
            s_v[sl] = (plsc.load_gather(b1_v, [pids, cb1])
                       + plsc.load_gather(b2_v, [pids, cb2]))
        pltpu.sync_copy(dots_v, dot_hbm.at[pl.ds(base, _BPW)])
        pltpu.sync_copy(s_v, s_hbm.at[pl.ds(base, _BPW)])

    return k(t1, t2, tab_t, bias_t)


def _tc_broadcast(dot, s):
    def body(d_ref, s_ref, o_ref):
        o_ref[...] = jnp.transpose(s_ref[...]) + d_ref[...]

    return pl.pallas_call(
        body,
        out_shape=jax.ShapeDtypeStruct((_B, _B), jnp.float32),
    )(dot.reshape(1, _B), s.reshape(1, _B))


def kernel(token1, token2, token_embedding, bias_embedding):
    t1 = token1.astype(jnp.int32)
    t2 = token2.astype(jnp.int32)
    dot, s = _sc_gather_dot(t1, t2, token_embedding.T,
                            bias_embedding.T)
    return _tc_broadcast(dot, s)


# speedup vs baseline: 12.3855x; 1.0113x over previous
import functools

import jax
import jax.numpy as jnp
from jax import lax
from jax.experimental import pallas as pl
from jax.experimental.pallas import tpu as pltpu
from jax.experimental.pallas import tpu_sc as plsc

_B = 1024
_F = 64
_W = 128
_NBUF = 7

_NC = 2
_NS = 16
_NW = _NC * _NS
_BPW = _B // _NW


def _sc_gather_dot(t1, t2, tab_t, bias_t):
    mesh = plsc.VectorSubcoreMesh(
        core_axis_name="c", subcore_axis_name="s",
        num_cores=_NC, num_subcores=_NS)

    @functools.partial(
        pl.kernel,
        mesh=mesh,
        out_type=[
            jax.ShapeDtypeStruct((_B,), jnp.float32),
            jax.ShapeDtypeStruct((_B,), jnp.float32),
        ],
        scratch_types=[
            pltpu.VMEM((_BPW,), jnp.int32),
            pltpu.VMEM((_BPW,), jnp.int32),
            pltpu.VMEM((_NBUF, _F, _W), jnp.float32),
            pltpu.VMEM((_NBUF, _F, _W), jnp.float32),
            pltpu.VMEM((_BPW, _W), jnp.float32),
            pltpu.VMEM((_BPW, _W), jnp.float32),
            pltpu.VMEM((_BPW,), jnp.float32),
            pltpu.VMEM((_BPW,), jnp.float32),
            pltpu.SemaphoreType.DMA((_NBUF,)),
            pltpu.SemaphoreType.DMA,
        ],
        compiler_params=pltpu.CompilerParams(needs_layout_passes=False),
    )
    def k(t1_hbm, t2_hbm, tab_hbm, bias_hbm, dot_hbm, s_hbm,
          idx1_v, idx2_v, e1_v, e2_v, b1_v, b2_v,
          dots_v, s_v, sem, bsem):
        wid = lax.axis_index("s") * _NC + lax.axis_index("c")
        base = wid * _BPW
        pltpu.sync_copy(t1_hbm.at[pl.ds(base, _BPW)], idx1_v)
        pltpu.sync_copy(t2_hbm.at[pl.ds(base, _BPW)], idx2_v)
        lanes = lax.iota(jnp.int32, 16)
        last = lanes == 15

        def scalar_at(ref, p):
            chunk = ref[pl.ds((p // 16) * 16, 16)]
            return lax.reduce_max(
                jnp.where(lanes == (p % 16), chunk, 0), axes=(0,))

        idx_scalars = [(scalar_at(idx1_v, p), scalar_at(idx2_v, p))
                       for p in range(_BPW)]

        def issue(p, slot):
            i1, i2 = idx_scalars[p]
            w1 = pl.multiple_of(jnp.bitwise_and(i1, -_W), _W)
            w2 = pl.multiple_of(jnp.bitwise_and(i2, -_W), _W)
            return (
                pltpu.async_copy(tab_hbm.at[:, pl.ds(w1, _W)],
                                 e1_v.at[slot], sem.at[slot]),
                pltpu.async_copy(tab_hbm.at[:, pl.ds(w2, _W)],
                                 e2_v.at[slot], sem.at[slot]),
            )

        pend = {}
        for p in range(min(_NBUF - 1, _BPW)):
            pend[p % _NBUF] = issue(p, p % _NBUF)
        bias_copies = []
        for p in range(_BPW):
            i1, i2 = idx_scalars[p]
            wb1 = pl.multiple_of(jnp.bitwise_and(i1, -_W), _W)
            wb2 = pl.multiple_of(jnp.bitwise_and(i2, -_W), _W)
            bias_copies.append(pltpu.async_copy(
                bias_hbm.at[0, pl.ds(wb1, _W)], b1_v.at[p], bsem))
            bias_copies.append(pltpu.async_copy(
                bias_hbm.at[0, pl.ds(wb2, _W)], b2_v.at[p], bsem))
        for p in range(_BPW):
            slot = p % _NBUF
            nxt = p + _NBUF - 1
            if nxt < _BPW:
                pend[nxt % _NBUF] = issue(nxt, nxt % _NBUF)
            for c in pend[slot]:
                c.wait()
            i1, i2 = idx_scalars[p]
            col1 = jnp.full((16,), jnp.bitwise_and(i1, _W - 1), jnp.int32)
            col2 = jnp.full((16,), jnp.bitwise_and(i2, _W - 1), jnp.int32)
            acc = None
            for g in range(_F // 16):
                fidx = jnp.full((16,), g * 16, jnp.int32) + lanes
                prod = (plsc.load_gather(e1_v.at[slot], [fidx, col1])
                        * plsc.load_gather(e2_v.at[slot], [fidx, col2]))
                acc = prod if acc is None else acc + prod
            csum = plsc.cumsum(acc)
            plsc.store_scatter(dots_v, [jnp.full((16,), p, jnp.int32)],
                               csum, mask=last)
        for c in bias_copies:
            c.wait()
        for g in range(_BPW // 16):
            sl = pl.ds(g * 16, 16)
            pids = jnp.full((16,), g * 16, jnp.int32) + lanes
            cb1 = jnp.bitwise_and(idx1_v[sl], _W - 1)
            cb2 = jnp.bitwise_and(idx2_v[sl], _W - 1)
            s_v[sl] = (plsc.load_gather(b1_v, [pids, cb1])
                       + plsc.load_gather(b2_v, [pids, cb2]))
        pltpu.sync_copy(dots_v, dot_hbm.at[pl.ds(base, _BPW)])
        pltpu.sync_copy(s_v, s_hbm.at[pl.ds(base, _BPW)])

    return k(t1, t2, tab_t, bias_t)


def _tc_broadcast(dot, s):
    def body(d_ref, s_ref, o_ref):
        o_ref[...] = jnp.transpose(s_ref[...]) + d_ref[...]

    return pl.pallas_call(
        body,
        out_shape=jax.ShapeDtypeStruct((_B, _B), jnp.float32),
    )(dot.reshape(1, _B), s.reshape(1, _B))


def kernel(token1, token2, token_embedding, bias_embedding):
    t1 = token1.astype(jnp.int32)
    t2 = token2.astype(jnp.int32)
    dot, s = _sc_gather_dot(t1, t2, token_embedding.T,
                            bias_embedding.T)
    return _tc_broadcast(dot, s)
